# baseline (speedup 1.0000x reference)
import dataclasses
import functools

import jax
import jax.numpy as jnp
from jax import lax
from jax.experimental import pallas as pl
from jax.experimental.pallas import tpu as pltpu
from jax.experimental.pallas import tpu_sc as plsc

_EPS = 1e-5
_LANES = 16
_CB = 256
_IDXW = 128
_PITCH = 65
_SL = 8


def _compiler_params():
    cp = pltpu.CompilerParams()
    for fld, val in (("needs_layout_passes", False),
                     ("use_tc_tiling_on_sc", False)):
        if fld in pltpu.CompilerParams.__dataclass_fields__:
            cp = dataclasses.replace(cp, **{fld: val})
    return cp


@functools.cache
def _make_transpose_kernel(V: int, D: int):
    info = plsc.get_sparse_core_info()
    NC, NS = info.num_cores, info.num_subcores
    NW = NC * NS
    n_tiles = (V + _IDXW - 1) // _IDXW
    Vpad = n_tiles * _IDXW
    n_slab = (n_tiles + NW - 1) // NW
    n_iter = (n_slab + 1) // 2
    na = D // _SL
    ncb = _IDXW // _LANES
    mesh = plsc.VectorSubcoreMesh(core_axis_name="c", subcore_axis_name="s")

    @functools.partial(
        pl.kernel,
        compiler_params=_compiler_params(),
        out_type=jax.ShapeDtypeStruct((Vpad, D), jnp.float32),
        mesh=mesh,
        scratch_types=[
            *[pltpu.VMEM((na, _SL, _IDXW), jnp.float32) for _ in range(2)],
            *[pltpu.VMEM((_IDXW, _PITCH), jnp.float32) for _ in range(2)],
            *[pltpu.VMEM((_IDXW, D), jnp.float32) for _ in range(2)],
            *[pltpu.SemaphoreType.DMA for _ in range(4)],
        ],
    )
    def tk(t4_hbm, out_hbm, *rest):
        sbuf = rest[0:2]
        obuf = rest[2:4]
        dbuf = rest[4:6]
        semi = rest[6:8]
        semo = rest[8:10]
        wid = lax.axis_index("s") * NC + lax.axis_index("c")

        def slab(k):
            return jnp.minimum(wid + k * NW, n_tiles - 1)

        def fire_i(k, st):
            pltpu.async_copy(t4_hbm.at[:, slab(k)], sbuf[st], semi[st])

        def wait_i(st):
            pltpu.make_async_copy(t4_hbm.at[:, 0], sbuf[st], semi[st]).wait()

        def fire_o(k, st):
            pltpu.async_copy(
                dbuf[st], out_hbm.at[pl.ds(slab(k) * _IDXW, _IDXW)], semo[st])

        def wait_o(st):
            pltpu.make_async_copy(
                dbuf[st], out_hbm.at[pl.ds(0, _IDXW)], semo[st]).wait()

        def compute(st):
            sb, ob, db = sbuf[st], obuf[st], dbuf[st]
            for a in range(na):
                for s in range(_SL):
                    j = a * _SL + s
                    cols = jnp.full((_LANES,), j, jnp.int32)
                    for cb in range(ncb):
                        x = sb[a, s, pl.ds(cb * _LANES, _LANES)]
                        rows = cb * _LANES + jnp.arange(_LANES,
                                                        dtype=jnp.int32)
                        plsc.store_scatter(ob, [rows, cols], x)
            for r in range(_IDXW):
                for i in range(D // _LANES):
                    db[r, pl.ds(i * _LANES, _LANES)] = (
                        ob[r, pl.ds(i * _LANES, _LANES)])

        fire_i(0, 0)
        fire_i(1, 1)

        @pl.loop(0, n_iter)
        def _t(t):
            for st in range(2):
                k = t * 2 + st
                wait_i(st)

                @pl.when(t >= 1)
                def _():
                    wait_o(st)
                compute(st)
                fire_o(k, st)
                fire_i(k + 2, st)

        for st in range(2):
            wait_i(st)
            wait_o(st)

    return tk


@functools.cache
def _make_sc_kernel(B: int, L: int, V: int, D: int):
    info = plsc.get_sparse_core_info()
    NC, NS = info.num_cores, info.num_subcores
    NW = NC * NS
    b_per_w = B // NW
    n_gather = _CB // _IDXW
    w_crows = b_per_w // _IDXW
    nv = D // _LANES
    nblk = _CB // _LANES
    cpchunk = _CB // _IDXW
    Vpad = ((V + _IDXW - 1) // _IDXW) * _IDXW
    mesh = plsc.VectorSubcoreMesh(core_axis_name="c", subcore_axis_name="s")

    @functools.partial(
        pl.kernel,
        compiler_params=_compiler_params(),
        out_type=jax.ShapeDtypeStruct((L, D // _SL, B // _IDXW, _SL, _IDXW),
                                      jnp.float32),
        mesh=mesh,
        scratch_types=[
            pltpu.VMEM((L * w_crows, _IDXW), jnp.int32),
            *[pltpu.VMEM((_CB, D), jnp.float32) for _ in range(2)],
            *[pltpu.VMEM((_CB, _PITCH), jnp.float32) for _ in range(2)],
            *[pltpu.VMEM((D // _SL, _CB // _IDXW, _SL, _IDXW), jnp.float32)
              for _ in range(2)],
            pltpu.VMEM((D,), jnp.float32),
            pltpu.VMEM((D,), jnp.float32),
            *[pltpu.SemaphoreType.DMA for _ in range(4)],
        ],
    )
    def k(idx_hbm, table_hbm, gamma_hbm, beta_hbm, out_hbm, idx_v, *rest):
        bufa = rest[0:2]
        bufp = rest[2:4]
        tbuf = rest[4:6]
        g_v, b_v = rest[6], rest[7]
        semg = rest[8:10]
        semo = rest[10:12]
        wid = lax.axis_index("s") * NC + lax.axis_index("c")
        for l in range(L):
            pltpu.async_copy(
                idx_hbm.at[l // _SL, pl.ds(wid * w_crows, w_crows), l % _SL],
                idx_v.at[pl.ds(l * w_crows, w_crows)],
                semg[0],
            )
        for l in range(L):
            pltpu.make_async_copy(
                idx_hbm.at[0, pl.ds(0, w_crows), 0],
                idx_v.at[pl.ds(0, w_crows)],
                semg[0],
            ).wait()
        pltpu.sync_copy(gamma_hbm, g_v)
        pltpu.sync_copy(beta_hbm, b_v)
        g = [g_v[pl.ds(i * _LANES, _LANES)] for i in range(nv)]
        b = [b_v[pl.ds(i * _LANES, _LANES)] for i in range(nv)]
        inv_d = jnp.float32(1.0 / D)

        def fire_g(l, h, st):
            for j in range(n_gather):
                pltpu.async_copy(
                    table_hbm.at[idx_v.at[l * w_crows + h * n_gather + j]],
                    bufa[st].at[pl.ds(j * _IDXW, _IDXW)],
                    semg[st],
                )

        def wait_g(st):
            pltpu.make_async_copy(
                table_hbm.at[pl.ds(0, _CB)], bufa[st], semg[st]
            ).wait()

        def fire_o(l, h, st):
            pltpu.async_copy(
                tbuf[st],
                out_hbm.at[l, :, pl.ds(wid * w_crows + h * cpchunk, cpchunk)],
                semo[st],
            )

        def wait_o(st):
            pltpu.make_async_copy(
                tbuf[st], out_hbm.at[0, :, pl.ds(0, cpchunk)], semo[st]
            ).wait()

        def compute(st):
            ba, bp, tb = bufa[st], bufp[st], tbuf[st]

            @plsc.parallel_loop(0, _CB, unroll=4)
            def _row(r):
                v = [ba[r, pl.ds(i * _LANES, _LANES)] for i in range(nv)]
                s = v[0]
                q = v[0] * v[0]
                for i in range(1, nv):
                    s = s + v[i]
                    q = q + v[i] * v[i]
                mean = jnp.sum(s) * inv_d
                var = jnp.sum(q) * inv_d - mean * mean
                x = var + jnp.float32(_EPS)
                seed = jnp.int32(0x5F3759DF) - (
                    lax.bitcast_convert_type(x, jnp.int32) >> 1)
                y = lax.bitcast_convert_type(seed, jnp.float32)
                nh = jnp.float32(-0.5) * x
                for _ in range(3):
                    y = y * (jnp.float32(1.5) + nh * y * y)
                mv = jnp.broadcast_to(mean, (_LANES,))
                yv = jnp.broadcast_to(y, (_LANES,))
                for i in range(nv):
                    bp[r, pl.ds(i * _LANES, _LANES)] = (
                        (v[i] - mv) * (yv * g[i]) + b[i]
                    )

            @plsc.parallel_loop(0, nblk)
            def _blk(blk):
                rows = blk * _LANES + jnp.arange(_LANES, dtype=jnp.int32)
                cb = blk // (_IDXW // _LANES)
                co = (blk % (_IDXW // _LANES)) * _LANES
                for j in range(D):
                    cols = jnp.full((_LANES,), j, jnp.int32)
                    xj = plsc.load_gather(bp, [rows, cols])
                    tb[j // _SL, cb, j % _SL, pl.ds(co, _LANES)] = xj

        fire_g(0, 0, 0)

        @pl.loop(0, L)
        def _t(t):
            for st in range(2):
                if st == 0:
                    fire_g(t, 1, 1)
                else:
                    @pl.when(t < L - 1)
                    def _():
                        fire_g(t + 1, 0, 0)
                wait_g(st)

                @pl.when(t >= 1)
                def _():
                    wait_o(st)
                compute(st)
                fire_o(t, st, st)

        for st in range(2):
            wait_o(st)

    return k


def kernel(input_ids, table, gamma, beta):
    B, L = input_ids.shape
    V, D = table.shape
    Lp = ((L + _SL - 1) // _SL) * _SL
    Vpad = ((V + _IDXW - 1) // _IDXW) * _IDXW
    ipad = jnp.pad(input_ids.astype(jnp.int32), ((0, 0), (0, Lp - L)))
    idx4 = (ipad.T.reshape(Lp // _SL, _SL, B // _IDXW, _IDXW)
            .transpose(0, 2, 1, 3))
    tpad = jnp.pad(table, ((0, Vpad - V), (0, 0)))
    t4 = (tpad.T.reshape(D // _SL, _SL, Vpad // _IDXW, _IDXW)
          .transpose(0, 2, 1, 3))
    t2 = _make_transpose_kernel(V, D)(t4)
    out5 = _make_sc_kernel(B, L, V, D)(idx4, t2, gamma, beta)
    out = (out5.transpose(0, 1, 3, 2, 4)
           .reshape(L, D, B)
           .transpose(2, 0, 1))
    return out


# speedup vs baseline: 1.3919x; 1.3919x over previous
import dataclasses
import functools

import jax
import jax.numpy as jnp
from jax import lax
from jax.experimental import pallas as pl
from jax.experimental.pallas import tpu as pltpu
from jax.experimental.pallas import tpu_sc as plsc

_EPS = 1e-5
_LANES = 16
_CB = 256
_IDXW = 128
_PITCH = 65
_SL = 8


@functools.cache
def _make_sc_kernel(B: int, L: int, V: int, D: int):
    info = plsc.get_sparse_core_info()
    NC, NS = info.num_cores, info.num_subcores
    NW = NC * NS
    Lp = ((L + _SL - 1) // _SL) * _SL
    b_per_w = B // NW
    halves = b_per_w // _CB
    n_gather = _CB // _IDXW
    w_crows = b_per_w // _IDXW
    nv = D // _LANES
    nblk = _CB // _LANES
    cpchunk = _CB // _IDXW
    mesh = plsc.VectorSubcoreMesh(core_axis_name="c", subcore_axis_name="s")
    cp = pltpu.CompilerParams()
    for fld, val in (("needs_layout_passes", False),
                     ("use_tc_tiling_on_sc", False)):
        if fld in pltpu.CompilerParams.__dataclass_fields__:
            cp = dataclasses.replace(cp, **{fld: val})

    @functools.partial(
        pl.kernel,
        compiler_params=cp,
        out_type=jax.ShapeDtypeStruct((L, D // _SL, B // _IDXW, _SL, _IDXW),
                                      jnp.float32),
        mesh=mesh,
        scratch_types=[
            pltpu.VMEM((L * w_crows, _IDXW), jnp.int32),
            *[pltpu.VMEM((_CB, D), jnp.float32) for _ in range(2)],
            *[pltpu.VMEM((_CB, _PITCH), jnp.float32) for _ in range(2)],
            *[pltpu.VMEM((D // _SL, cpchunk, _SL, _IDXW), jnp.float32)
              for _ in range(2)],
            pltpu.VMEM((D,), jnp.float32),
            pltpu.VMEM((D,), jnp.float32),
            *[pltpu.SemaphoreType.DMA for _ in range(4)],
        ],
    )
    def k(idx_hbm, table_hbm, gamma_hbm, beta_hbm, out_hbm, idx_v, *rest):
        bufa = rest[0:2]
        bufp = rest[2:4]
        tbuf = rest[4:6]
        g_v, b_v = rest[6], rest[7]
        semg = rest[8:10]
        semo = rest[10:12]
        wid = lax.axis_index("s") * NC + lax.axis_index("c")
        for l in range(L):
            pltpu.async_copy(
                idx_hbm.at[l // _SL, pl.ds(wid * w_crows, w_crows), l % _SL],
                idx_v.at[pl.ds(l * w_crows, w_crows)],
                semg[0],
            )
        for l in range(L):
            pltpu.make_async_copy(
                idx_hbm.at[0, pl.ds(0, w_crows), 0],
                idx_v.at[pl.ds(0, w_crows)],
                semg[0],
            ).wait()
        pltpu.sync_copy(gamma_hbm, g_v)
        pltpu.sync_copy(beta_hbm, b_v)
        g = [g_v[pl.ds(i * _LANES, _LANES)] for i in range(nv)]
        b = [b_v[pl.ds(i * _LANES, _LANES)] for i in range(nv)]
        inv_d = jnp.float32(1.0 / D)

        def fire_g(l, h, st):
            for j in range(n_gather):
                pltpu.async_copy(
                    table_hbm.at[idx_v.at[l * w_crows + h * n_gather + j]],
                    bufa[st].at[pl.ds(j * _IDXW, _IDXW)],
                    semg[st],
                )

        def wait_g(st):
            pltpu.make_async_copy(
                table_hbm.at[pl.ds(0, _CB)], bufa[st], semg[st]
            ).wait()

        def fire_o(l, h, st):
            pltpu.async_copy(
                tbuf[st],
                out_hbm.at[l, :, pl.ds(wid * w_crows + h * cpchunk, cpchunk)],
                semo[st],
            )

        def wait_o(st):
            pltpu.make_async_copy(
                tbuf[st], out_hbm.at[0, :, pl.ds(0, cpchunk)], semo[st]
            ).wait()

        def compute(st):
            ba, bp, tb = bufa[st], bufp[st], tbuf[st]

            @plsc.parallel_loop(0, _CB, unroll=4)
            def _row(r):
                v = [ba[r, pl.ds(i * _LANES, _LANES)] for i in range(nv)]
                s = v[0]
                q = v[0] * v[0]
                for i in range(1, nv):
                    s = s + v[i]
                    q = q + v[i] * v[i]
                mean = jnp.sum(s) * inv_d
                var = jnp.sum(q) * inv_d - mean * mean
                x = var + jnp.float32(_EPS)
                seed = jnp.int32(0x5F3759DF) - (
                    lax.bitcast_convert_type(x, jnp.int32) >> 1)
                y = lax.bitcast_convert_type(seed, jnp.float32)
                nh = jnp.float32(-0.5) * x
                for _ in range(3):
                    y = y * (jnp.float32(1.5) + nh * y * y)
                mv = jnp.broadcast_to(mean, (_LANES,))
                yv = jnp.broadcast_to(y, (_LANES,))
                for i in range(nv):
                    bp[r, pl.ds(i * _LANES, _LANES)] = (
                        (v[i] - mv) * (yv * g[i]) + b[i]
                    )

            @plsc.parallel_loop(0, nblk, unroll=2)
            def _blk(blk):
                rows = blk * _LANES + jnp.arange(_LANES, dtype=jnp.int32)
                cb = blk // (_IDXW // _LANES)
                co = (blk % (_IDXW // _LANES)) * _LANES
                for j in range(D):
                    cols = jnp.full((_LANES,), j, jnp.int32)
                    xj = plsc.load_gather(bp, [rows, cols])
                    tb[j // _SL, cb, j % _SL, pl.ds(co, _LANES)] = xj

        fire_g(0, 0, 0)

        @pl.loop(0, L)
        def _t(t):
            for st in range(2):
                if st == 0:
                    fire_g(t, 1, 1)
                else:
                    @pl.when(t < L - 1)
                    def _():
                        fire_g(t + 1, 0, 0)
                wait_g(st)

                @pl.when(t >= 1)
                def _():
                    wait_o(st)
                compute(st)
                fire_o(t, st, st)

        for st in range(2):
            wait_o(st)

    return k


def kernel(input_ids, table, gamma, beta):
    B, L = input_ids.shape
    V, D = table.shape
    Lp = ((L + _SL - 1) // _SL) * _SL
    padded = jnp.pad(input_ids.astype(jnp.int32), ((0, 0), (0, Lp - L)))
    idx4 = (padded.T.reshape(Lp // _SL, _SL, B // _IDXW, _IDXW)
            .transpose(0, 2, 1, 3))
    out5 = _make_sc_kernel(B, L, V, D)(idx4, table, gamma, beta)
    out = (out5.transpose(0, 1, 3, 2, 4)
           .reshape(L, D, B)
           .transpose(2, 0, 1))
    return out


# speedup vs baseline: 1.4666x; 1.0537x over previous
import dataclasses
import functools

import jax
import jax.numpy as jnp
from jax import lax
from jax.experimental import pallas as pl
from jax.experimental.pallas import tpu as pltpu
from jax.experimental.pallas import tpu_sc as plsc

_EPS = 1e-5
_LANES = 16
_CB = 256
_IDXW = 128
_PITCH = 65
_SL = 8


@functools.cache
def _make_sc_kernel(B: int, L: int, V: int, D: int):
    info = plsc.get_sparse_core_info()
    NC, NS = info.num_cores, info.num_subcores
    NW = NC * NS
    Lp = ((L + _SL - 1) // _SL) * _SL
    b_per_w = B // NW
    halves = b_per_w // _CB
    n_gather = _CB // _IDXW
    w_crows = b_per_w // _IDXW
    nv = D // _LANES
    nblk = _CB // _LANES
    cpchunk = _CB // _IDXW
    mesh = plsc.VectorSubcoreMesh(core_axis_name="c", subcore_axis_name="s")
    cp = pltpu.CompilerParams()
    for fld, val in (("needs_layout_passes", False),
                     ("use_tc_tiling_on_sc", False)):
        if fld in pltpu.CompilerParams.__dataclass_fields__:
            cp = dataclasses.replace(cp, **{fld: val})

    @functools.partial(
        pl.kernel,
        compiler_params=cp,
        out_type=jax.ShapeDtypeStruct((L, D // _SL, B // _IDXW, _SL, _IDXW),
                                      jnp.float32),
        mesh=mesh,
        scratch_types=[
            pltpu.VMEM((L * w_crows, _IDXW), jnp.int32),
            *[pltpu.VMEM((_CB, D), jnp.float32) for _ in range(2)],
            *[pltpu.VMEM((_CB, _PITCH), jnp.float32) for _ in range(2)],
            *[pltpu.VMEM((D // _SL, cpchunk, _SL, _IDXW), jnp.float32)
              for _ in range(2)],
            pltpu.VMEM((D,), jnp.float32),
            pltpu.VMEM((D,), jnp.float32),
            *[pltpu.SemaphoreType.DMA for _ in range(4)],
        ],
    )
    def k(idx_hbm, table_hbm, gamma_hbm, beta_hbm, out_hbm, idx_v, *rest):
        bufa = rest[0:2]
        bufp = rest[2:4]
        tbuf = rest[4:6]
        g_v, b_v = rest[6], rest[7]
        semg = rest[8:10]
        semo = rest[10:12]
        wid = lax.axis_index("s") * NC + lax.axis_index("c")
        for l in range(L):
            pltpu.async_copy(
                idx_hbm.at[l // _SL, pl.ds(wid * w_crows, w_crows), l % _SL],
                idx_v.at[pl.ds(l * w_crows, w_crows)],
                semg[0],
            )
        for l in range(L):
            pltpu.make_async_copy(
                idx_hbm.at[0, pl.ds(0, w_crows), 0],
                idx_v.at[pl.ds(0, w_crows)],
                semg[0],
            ).wait()
        pltpu.sync_copy(gamma_hbm, g_v)
        pltpu.sync_copy(beta_hbm, b_v)
        g = [g_v[pl.ds(i * _LANES, _LANES)] for i in range(nv)]
        b = [b_v[pl.ds(i * _LANES, _LANES)] for i in range(nv)]
        inv_d = jnp.float32(1.0 / D)

        def fire_g(l, h, st):
            for j in range(n_gather):
                pltpu.async_copy(
                    table_hbm.at[idx_v.at[l * w_crows + h * n_gather + j]],
                    bufa[st].at[pl.ds(j * _IDXW, _IDXW)],
                    semg[st],
                )

        def wait_g(st):
            pltpu.make_async_copy(
                table_hbm.at[pl.ds(0, _CB)], bufa[st], semg[st]
            ).wait()

        def fire_o(l, h, st):
            pltpu.async_copy(
                tbuf[st],
                out_hbm.at[l, :, pl.ds(wid * w_crows + h * cpchunk, cpchunk)],
                semo[st],
            )

        def wait_o(st):
            pltpu.make_async_copy(
                tbuf[st], out_hbm.at[0, :, pl.ds(0, cpchunk)], semo[st]
            ).wait()

        def compute(st):
            ba, bp, tb = bufa[st], bufp[st], tbuf[st]

            @plsc.parallel_loop(0, _CB, unroll=8)
            def _row(r):
                v = [ba[r, pl.ds(i * _LANES, _LANES)] for i in range(nv)]
                s = v[0]
                q = v[0] * v[0]
                for i in range(1, nv):
                    s = s + v[i]
                    q = q + v[i] * v[i]
                mean = jnp.sum(s) * inv_d
                var = jnp.sum(q) * inv_d - mean * mean
                x = var + jnp.float32(_EPS)
                seed = jnp.int32(0x5F3759DF) - (
                    lax.bitcast_convert_type(x, jnp.int32) >> 1)
                y = lax.bitcast_convert_type(seed, jnp.float32)
                nh = jnp.float32(-0.5) * x
                for _ in range(3):
                    y = y * (jnp.float32(1.5) + nh * y * y)
                mv = jnp.broadcast_to(mean, (_LANES,))
                yv = jnp.broadcast_to(y, (_LANES,))
                for i in range(nv):
                    bp[r, pl.ds(i * _LANES, _LANES)] = (
                        (v[i] - mv) * (yv * g[i]) + b[i]
                    )

            @plsc.parallel_loop(0, nblk)
            def _blk(blk):
                rows = blk * _LANES + jnp.arange(_LANES, dtype=jnp.int32)
                cb = blk // (_IDXW // _LANES)
                co = (blk % (_IDXW // _LANES)) * _LANES
                for j in range(D):
                    cols = jnp.full((_LANES,), j, jnp.int32)
                    xj = plsc.load_gather(bp, [rows, cols])
                    tb[j // _SL, cb, j % _SL, pl.ds(co, _LANES)] = xj

        fire_g(0, 0, 0)

        @pl.loop(0, L)
        def _t(t):
            for st in range(2):
                if st == 0:
                    fire_g(t, 1, 1)
                else:
                    @pl.when(t < L - 1)
                    def _():
                        fire_g(t + 1, 0, 0)
                wait_g(st)

                @pl.when(t >= 1)
                def _():
                    wait_o(st)
                compute(st)
                fire_o(t, st, st)

        for st in range(2):
            wait_o(st)

    return k


def kernel(input_ids, table, gamma, beta):
    B, L = input_ids.shape
    V, D = table.shape
    Lp = ((L + _SL - 1) // _SL) * _SL
    padded = jnp.pad(input_ids.astype(jnp.int32), ((0, 0), (0, Lp - L)))
    idx4 = (padded.T.reshape(Lp // _SL, _SL, B // _IDXW, _IDXW)
            .transpose(0, 2, 1, 3))
    out5 = _make_sc_kernel(B, L, V, D)(idx4, table, gamma, beta)
    out = (out5.transpose(0, 1, 3, 2, 4)
           .reshape(L, D, B)
           .transpose(2, 0, 1))
    return out


# speedup vs baseline: 1.4934x; 1.0183x over previous
import dataclasses
import functools

import jax
import jax.numpy as jnp
from jax import lax
from jax.experimental import pallas as pl
from jax.experimental.pallas import tpu as pltpu
from jax.experimental.pallas import tpu_sc as plsc

_EPS = 1e-5
_LANES = 16
_CB = 256
_IDXW = 128
_PITCH = 65
_SL = 8


@functools.cache
def _make_sc_kernel(B: int, L: int, V: int, D: int):
    info = plsc.get_sparse_core_info()
    NC, NS = info.num_cores, info.num_subcores
    NW = NC * NS
    Lp = ((L + _SL - 1) // _SL) * _SL
    b_per_w = B // NW
    halves = b_per_w // _CB
    n_gather = _CB // _IDXW
    w_crows = b_per_w // _IDXW
    nv = D // _LANES
    nblk = _CB // _LANES
    cpchunk = _CB // _IDXW
    mesh = plsc.VectorSubcoreMesh(core_axis_name="c", subcore_axis_name="s")
    cp = pltpu.CompilerParams()
    for fld, val in (("needs_layout_passes", False),
                     ("use_tc_tiling_on_sc", False)):
        if fld in pltpu.CompilerParams.__dataclass_fields__:
            cp = dataclasses.replace(cp, **{fld: val})

    @functools.partial(
        pl.kernel,
        compiler_params=cp,
        out_type=jax.ShapeDtypeStruct((L, D // _SL, B // _IDXW, _SL, _IDXW),
                                      jnp.float32),
        mesh=mesh,
        scratch_types=[
            pltpu.VMEM((L * w_crows, _IDXW), jnp.int32),
            *[pltpu.VMEM((_CB, D), jnp.float32) for _ in range(2)],
            *[pltpu.VMEM((_CB, _PITCH), jnp.float32) for _ in range(2)],
            *[pltpu.VMEM((D // _SL, cpchunk, _SL, _IDXW), jnp.float32)
              for _ in range(2)],
            pltpu.VMEM((D,), jnp.float32),
            pltpu.VMEM((D,), jnp.float32),
            *[pltpu.SemaphoreType.DMA for _ in range(4)],
        ],
    )
    def k(idx_hbm, table_hbm, gamma_hbm, beta_hbm, out_hbm, idx_v, *rest):
        bufa = rest[0:2]
        bufp = rest[2:4]
        tbuf = rest[4:6]
        g_v, b_v = rest[6], rest[7]
        semg = rest[8:10]
        semo = rest[10:12]
        wid = lax.axis_index("s") * NC + lax.axis_index("c")
        for l in range(L):
            pltpu.async_copy(
                idx_hbm.at[l // _SL, pl.ds(wid * w_crows, w_crows), l % _SL],
                idx_v.at[pl.ds(l * w_crows, w_crows)],
                semg[0],
            )
        for l in range(L):
            pltpu.make_async_copy(
                idx_hbm.at[0, pl.ds(0, w_crows), 0],
                idx_v.at[pl.ds(0, w_crows)],
                semg[0],
            ).wait()
        pltpu.sync_copy(gamma_hbm, g_v)
        pltpu.sync_copy(beta_hbm, b_v)
        g = [g_v[pl.ds(i * _LANES, _LANES)] for i in range(nv)]
        b = [b_v[pl.ds(i * _LANES, _LANES)] for i in range(nv)]
        inv_d = jnp.float32(1.0 / D)

        def fire_g(l, h, st):
            for j in range(n_gather):
                pltpu.async_copy(
                    table_hbm.at[idx_v.at[l * w_crows + h * n_gather + j]],
                    bufa[st].at[pl.ds(j * _IDXW, _IDXW)],
                    semg[st],
                )

        def wait_g(st):
            pltpu.make_async_copy(
                table_hbm.at[pl.ds(0, _CB)], bufa[st], semg[st]
            ).wait()

        def fire_o(l, h, st):
            pltpu.async_copy(
                tbuf[st],
                out_hbm.at[l, :, pl.ds(wid * w_crows + h * cpchunk, cpchunk)],
                semo[st],
            )

        def wait_o(st):
            pltpu.make_async_copy(
                tbuf[st], out_hbm.at[0, :, pl.ds(0, cpchunk)], semo[st]
            ).wait()

        def compute(st):
            ba, bp, tb = bufa[st], bufp[st], tbuf[st]

            @plsc.parallel_loop(0, _CB, unroll=4)
            def _row(r):
                v = [ba[r, pl.ds(i * _LANES, _LANES)] for i in range(nv)]
                s = v[0]
                q = v[0] * v[0]
                for i in range(1, nv):
                    s = s + v[i]
                    q = q + v[i] * v[i]
                mean = jnp.sum(s) * inv_d
                var = jnp.sum(q) * inv_d - mean * mean
                x = var + jnp.float32(_EPS)
                seed = jnp.int32(0x5F3759DF) - (
                    lax.bitcast_convert_type(x, jnp.int32) >> 1)
                y = lax.bitcast_convert_type(seed, jnp.float32)
                nh = jnp.float32(-0.5) * x
                for _ in range(3):
                    y = y * (jnp.float32(1.5) + nh * y * y)
                mv = jnp.broadcast_to(mean, (_LANES,))
                yv = jnp.broadcast_to(y, (_LANES,))
                for i in range(nv):
                    bp[r, pl.ds(i * _LANES, _LANES)] = (
                        (v[i] - mv) * (yv * g[i]) + b[i]
                    )

            @plsc.parallel_loop(0, nblk)
            def _blk(blk):
                rows = blk * _LANES + jnp.arange(_LANES, dtype=jnp.int32)
                cb = blk // (_IDXW // _LANES)
                co = (blk % (_IDXW // _LANES)) * _LANES
                for j in range(D):
                    cols = jnp.full((_LANES,), j, jnp.int32)
                    xj = plsc.load_gather(bp, [rows, cols])
                    tb[j // _SL, cb, j % _SL, pl.ds(co, _LANES)] = xj

        fire_g(0, 0, 0)

        @pl.loop(0, L)
        def _t(t):
            for st in range(2):
                if st == 0:
                    fire_g(t, 1, 1)
                else:
                    @pl.when(t < L - 1)
                    def _():
                        fire_g(t + 1, 0, 0)
                wait_g(st)

                @pl.when(t >= 1)
                def _():
                    wait_o(st)
                compute(st)
                fire_o(t, st, st)

        for st in range(2):
            wait_o(st)

    return k


def kernel(input_ids, table, gamma, beta):
    B, L = input_ids.shape
    V, D = table.shape
    Lp = ((L + _SL - 1) // _SL) * _SL
    padded = jnp.pad(input_ids.astype(jnp.int32), ((0, 0), (0, Lp - L)))
    idx4 = (padded.T.reshape(Lp // _SL, _SL, B // _IDXW, _IDXW)
            .transpose(0, 2, 1, 3))
    out5 = _make_sc_kernel(B, L, V, D)(idx4, table, gamma, beta)
    out = (out5.transpose(0, 1, 3, 2, 4)
           .reshape(L, D, B)
           .transpose(2, 0, 1))
    return out
